# baseline (speedup 1.0000x reference)
import functools

import jax
import jax.numpy as jnp
from jax import lax
from jax.experimental import pallas as pl
from jax.experimental.pallas import tpu as pltpu
from jax.experimental.pallas import tpu_sc as plsc

S = 2048
DM = 768
NH = 12
DH = 64
DR = 32
KS = 32
LW = 16
DCKV = 256
DCQ = 384
DCAT = DCKV + DR
DPAD = 384

BA = 256
BD = 128
SCALE = (DH + DR) ** -0.5
INT_MIN = -(2 ** 31)


def _softplus(x):
    return jnp.maximum(x, 0.0) + jnp.log(1.0 + jnp.exp(-jnp.abs(x)))


def _proj_body(x_ref, wdkv_ref, wkr_ref, wdq_ref, wuq_ref, wqr_ref, wuk_ref,
               wqi_ref, wki_ref, cosq_ref, sinq_ref,
               ckv_ref, xkr_ref, qi_ref, ki_ref, qabs_ref, qr_ref):
    xb = x_ref[...]
    f32 = jnp.float32
    ckv_ref[...] = jnp.dot(xb, wdkv_ref[...], preferred_element_type=f32)
    xkr_ref[...] = jnp.dot(xb, wkr_ref[...], preferred_element_type=f32)
    qi_ref[...] = jnp.dot(xb, wqi_ref[...], preferred_element_type=f32)
    ki_ref[...] = jnp.dot(xb, wki_ref[...], preferred_element_type=f32)
    cq = jnp.dot(xb, wdq_ref[...], preferred_element_type=f32)
    qc = jnp.dot(cq, wuq_ref[...], preferred_element_type=f32)
    wuk = wuk_ref[...]
    cos = cosq_ref[...]
    sin = sinq_ref[...]
    for h in range(NH):
        qch = qc[:, h * DH:(h + 1) * DH]
        qabs_ref[:, h, :] = lax.dot_general(
            qch, wuk[:, h * DH:(h + 1) * DH],
            (((1,), (1,)), ((), ())), preferred_element_type=f32)
        qrh = jnp.dot(cq, wqr_ref[:, h * DR:(h + 1) * DR],
                      preferred_element_type=f32)
        sp = _softplus(qrh)
        mu1 = sp[:, :DR // 2]
        mu2 = sp[:, DR // 2:]
        qr_ref[:, h, :] = jnp.concatenate(
            [mu1 * cos - mu2 * sin, mu1 * sin + mu2 * cos], axis=1)


def _topk_body(qi_ref, ki_ref, widx_ref, idx_ref, *, blk, W):
    i = blk
    f32 = jnp.float32
    ki = ki_ref[...]
    acc = jnp.zeros((BA, W), f32)
    wb = widx_ref[...].astype(jnp.bfloat16).astype(f32)
    for h in range(NH):
        sh = lax.dot_general(qi_ref[:, h * DH:(h + 1) * DH], ki,
                             (((1,), (1,)), ((), ())),
                             preferred_element_type=f32)
        rb = jnp.maximum(sh, 0.0).astype(jnp.bfloat16).astype(f32)
        acc = acc + rb * wb[0:1, h:h + 1]

    qpos = i * BA + lax.broadcasted_iota(jnp.int32, (BA, 1), 0)
    kio = lax.broadcasted_iota(jnp.int32, (BA, W), 1)
    base = jnp.maximum(qpos - (LW - 1), 0)
    reserved = (kio >= base) & (kio <= base + (LW - 1))
    acc = jnp.where(kio > qpos, -jnp.inf, acc)
    bits = lax.bitcast_convert_type(acc, jnp.int32)
    u = jnp.where(bits < 0, bits ^ 0x7FFFFFFF, bits)
    u = jnp.where(reserved, INT_MIN, u)

    slot = lax.broadcasted_iota(jnp.int32, (BA, LW), 1)

    def body(p, carry):
        u, sel = carry
        m = jnp.max(u, axis=1, keepdims=True)
        pos = jnp.min(jnp.where(u == m, kio, W), axis=1, keepdims=True)
        u = jnp.where(kio == pos, INT_MIN, u)
        sel = jnp.where(slot == p, pos, sel)
        return u, sel

    _, sel = lax.fori_loop(0, KS - LW, body,
                           (u, jnp.zeros((BA, LW), jnp.int32)))
    idx_ref[:, :LW] = base + slot
    idx_ref[:, LW:] = sel


@functools.lru_cache(maxsize=None)
def _make_sc_gather():
    info = plsc.get_sparse_core_info()
    nw = info.num_cores * info.num_subcores
    chunk = 128
    b_per_w = (S * KS) // nw
    nchunks = b_per_w // chunk
    mesh = plsc.VectorSubcoreMesh(core_axis_name="c", subcore_axis_name="s")

    @functools.partial(
        pl.kernel, mesh=mesh,
        out_type=jax.ShapeDtypeStruct((S * KS, DCAT), jnp.float32),
        scratch_types=[
            pltpu.VMEM((chunk,), jnp.int32),
            pltpu.VMEM((chunk, DCAT), jnp.float32),
            pltpu.SemaphoreType.DMA,
        ],
        compiler_params=pltpu.CompilerParams(use_tc_tiling_on_sc=False),
    )
    def gather(table_hbm, idx_hbm, out_hbm, idx_v, rows_v, sem):
        wid = lax.axis_index("s") * info.num_cores + lax.axis_index("c")
        base = wid * b_per_w

        def body(c, carry):
            off = base + c * chunk
            pltpu.sync_copy(idx_hbm.at[pl.ds(off, chunk)], idx_v)
            pltpu.async_copy(table_hbm.at[idx_v], rows_v, sem).wait()
            pltpu.sync_copy(rows_v, out_hbm.at[pl.ds(off, chunk)])
            return carry

        lax.fori_loop(0, nchunks, body, 0)

    return gather


def _sc_gather(cat, idx_flat):
    return _make_sc_gather()(cat, idx_flat)


def _attn_body(qabs_ref, qr_ref, sel_ref, wuv_ref, wout_ref,
               cosj_ref, sinj_ref, out_ref):
    f32 = jnp.float32
    sel = sel_ref[...].reshape(BD, KS, DCAT)
    ckv = sel[..., :DCKV]
    kr = sel[..., DCKV:DCAT]
    sp = _softplus(kr)
    mu1 = sp[..., :DR // 2]
    mu2 = sp[..., DR // 2:]
    cosj = cosj_ref[...][None]
    sinj = sinj_ref[...][None]
    keyr = jnp.concatenate(
        [mu1 * cosj - mu2 * sinj, mu1 * sinj + mu2 * cosj], axis=-1)
    wuv = wuv_ref[...]
    qabs = qabs_ref[...]
    qrr = qr_ref[...]
    s_c = lax.dot_general(qabs, ckv, (((2,), (2,)), ((0,), (0,))),
                          preferred_element_type=f32)
    s_r = lax.dot_general(qrr, keyr, (((2,), (2,)), ((0,), (0,))),
                          preferred_element_type=f32)
    s = (s_c + s_r) * SCALE
    m = jnp.max(s, axis=-1, keepdims=True)
    e = jnp.exp(s - m)
    a = e / jnp.sum(e, axis=-1, keepdims=True)
    v = lax.dot_general(a, ckv, (((2,), (1,)), ((0,), (0,))),
                        preferred_element_type=f32)
    outs = []
    for h in range(NH):
        outs.append(jnp.dot(v[:, h, :], wuv[:, h * DH:(h + 1) * DH],
                            preferred_element_type=f32))
    o = jnp.concatenate(outs, axis=1)
    out_ref[...] = jnp.dot(o, wout_ref[...], preferred_element_type=f32)


def _full(shape, dtype=jnp.float32):
    return pl.BlockSpec(shape, lambda i: (0,) * len(shape))


def kernel(x, w_dkv, w_uk, w_uv, w_dq, w_uq, w_qr, w_kr, w_out,
           wq_idx, wk_idx, w_idx, raw_delta):
    f32 = jnp.float32
    x2 = x.reshape(S, DM)

    theta = 1.0 / (10000.0 ** (2.0 * jnp.arange(0, DR // 2) / DR))
    delta = -2.0 * jnp.pi * jax.nn.sigmoid(raw_delta)
    angq = jnp.arange(S)[:, None] * theta[None, :] + delta[None, :]
    cosq, sinq = jnp.cos(angq).astype(f32), jnp.sin(angq).astype(f32)
    angj = jnp.arange(KS)[:, None] * theta[None, :] + delta[None, :]
    cosj, sinj = jnp.cos(angj).astype(f32), jnp.sin(angj).astype(f32)

    grid_a = S // BA
    ckv, xkr, qi, ki, qabs, qr = pl.pallas_call(
        _proj_body,
        grid=(grid_a,),
        in_specs=[
            pl.BlockSpec((BA, DM), lambda i: (i, 0)),
            _full((DM, DCKV)), _full((DM, DR)), _full((DM, DCQ)),
            _full((DCQ, NH * DH)), _full((DCQ, NH * DR)),
            _full((DCKV, NH * DH)), _full((DM, NH * DH)), _full((DM, DH)),
            pl.BlockSpec((BA, DR // 2), lambda i: (i, 0)),
            pl.BlockSpec((BA, DR // 2), lambda i: (i, 0)),
        ],
        out_specs=[
            pl.BlockSpec((BA, DCKV), lambda i: (i, 0)),
            pl.BlockSpec((BA, DR), lambda i: (i, 0)),
            pl.BlockSpec((BA, DM), lambda i: (i, 0)),
            pl.BlockSpec((BA, DH), lambda i: (i, 0)),
            pl.BlockSpec((BA, NH, DCKV), lambda i: (i, 0, 0)),
            pl.BlockSpec((BA, NH, DR), lambda i: (i, 0, 0)),
        ],
        out_shape=[
            jax.ShapeDtypeStruct((S, DCKV), f32),
            jax.ShapeDtypeStruct((S, DR), f32),
            jax.ShapeDtypeStruct((S, DM), f32),
            jax.ShapeDtypeStruct((S, DH), f32),
            jax.ShapeDtypeStruct((S, NH, DCKV), f32),
            jax.ShapeDtypeStruct((S, NH, DR), f32),
        ],
    )(x2, w_dkv, w_kr, w_dq, w_uq, w_qr, w_uk, wq_idx, wk_idx, cosq, sinq)

    idx_parts = []
    for blk in range(S // BA):
        W = BA * (blk + 1)
        idx_parts.append(pl.pallas_call(
            functools.partial(_topk_body, blk=blk, W=W),
            grid=(1,),
            in_specs=[
                pl.BlockSpec((BA, DM), lambda i, b=blk: (b, 0)),
                pl.BlockSpec((W, DH), lambda i: (0, 0)),
                _full((1, NH)),
            ],
            out_specs=pl.BlockSpec((BA, KS), lambda i: (0, 0)),
            out_shape=jax.ShapeDtypeStruct((BA, KS), jnp.int32),
        )(qi, ki, w_idx.reshape(1, NH)))
    idx = jnp.concatenate(idx_parts, axis=0)

    cat = jnp.concatenate([ckv, xkr], axis=1)
    sel = _sc_gather(cat, idx.reshape(S * KS))

    out = pl.pallas_call(
        _attn_body,
        grid=(S // BD,),
        in_specs=[
            pl.BlockSpec((BD, NH, DCKV), lambda i: (i, 0, 0)),
            pl.BlockSpec((BD, NH, DR), lambda i: (i, 0, 0)),
            pl.BlockSpec((BD * KS, DCAT), lambda i: (i, 0)),
            _full((DCKV, NH * DH)), _full((NH * DH, DM)),
            _full((KS, DR // 2)), _full((KS, DR // 2)),
        ],
        out_specs=pl.BlockSpec((BD, DM), lambda i: (i, 0)),
        out_shape=jax.ShapeDtypeStruct((S, DM), f32),
    )(qabs, qr, sel, w_uv, w_out, cosj, sinj)

    return out.reshape(1, S, DM)


# speedup vs baseline: 1.2314x; 1.2314x over previous
import functools

import jax
import jax.numpy as jnp
from jax import lax
from jax.experimental import pallas as pl
from jax.experimental.pallas import tpu as pltpu
from jax.experimental.pallas import tpu_sc as plsc

S = 2048
DM = 768
NH = 12
DH = 64
DR = 32
KS = 32
LW = 16
DCKV = 256
DCQ = 384
DCAT = DCKV + DR
DPAD = 384

BA = 256
BD = 128
SCALE = (DH + DR) ** -0.5
INT_MIN = -(2 ** 31)


def _softplus(x):
    return jnp.maximum(x, 0.0) + jnp.log(1.0 + jnp.exp(-jnp.abs(x)))


def _proj_body(x_ref, wdkv_ref, wkr_ref, wdq_ref, wuq_ref, wqr_ref, wuk_ref,
               wqi_ref, wki_ref, cosq_ref, sinq_ref,
               ckv_ref, xkr_ref, qi_ref, ki_ref, qabs_ref, qr_ref):
    xb = x_ref[...]
    f32 = jnp.float32
    ckv_ref[...] = jnp.dot(xb, wdkv_ref[...], preferred_element_type=f32)
    xkr_ref[...] = jnp.dot(xb, wkr_ref[...], preferred_element_type=f32)
    qi_ref[...] = jnp.dot(xb, wqi_ref[...], preferred_element_type=f32)
    ki_ref[...] = jnp.dot(xb, wki_ref[...], preferred_element_type=f32)
    cq = jnp.dot(xb, wdq_ref[...], preferred_element_type=f32)
    qc = jnp.dot(cq, wuq_ref[...], preferred_element_type=f32)
    wuk = wuk_ref[...]
    cos = cosq_ref[...]
    sin = sinq_ref[...]
    for h in range(NH):
        qch = qc[:, h * DH:(h + 1) * DH]
        qabs_ref[:, h, :] = lax.dot_general(
            qch, wuk[:, h * DH:(h + 1) * DH],
            (((1,), (1,)), ((), ())), preferred_element_type=f32)
        qrh = jnp.dot(cq, wqr_ref[:, h * DR:(h + 1) * DR],
                      preferred_element_type=f32)
        sp = _softplus(qrh)
        mu1 = sp[:, :DR // 2]
        mu2 = sp[:, DR // 2:]
        qr_ref[:, h, :] = jnp.concatenate(
            [mu1 * cos - mu2 * sin, mu1 * sin + mu2 * cos], axis=1)


def _topk_body(qi_ref, ki_ref, widx_ref, idx_ref, *, blk, W):
    i = blk
    f32 = jnp.float32
    ki = ki_ref[...]
    acc = jnp.zeros((BA, W), f32)
    wb = widx_ref[...].astype(jnp.bfloat16).astype(f32)
    for h in range(NH):
        sh = lax.dot_general(qi_ref[:, h * DH:(h + 1) * DH], ki,
                             (((1,), (1,)), ((), ())),
                             preferred_element_type=f32)
        rb = jnp.maximum(sh, 0.0).astype(jnp.bfloat16).astype(f32)
        acc = acc + rb * wb[0:1, h:h + 1]

    qpos = i * BA + lax.broadcasted_iota(jnp.int32, (BA, 1), 0)
    kio = lax.broadcasted_iota(jnp.int32, (BA, W), 1)
    base = jnp.maximum(qpos - (LW - 1), 0)
    reserved = (kio >= base) & (kio <= base + (LW - 1))
    acc = jnp.where(kio > qpos, -jnp.inf, acc)
    bits = lax.bitcast_convert_type(acc, jnp.int32)
    u = jnp.where(bits < 0, bits ^ 0x7FFFFFFF, bits)
    u = jnp.where(reserved, INT_MIN, u)

    slot = lax.broadcasted_iota(jnp.int32, (BA, LW), 1)

    def body(p, carry):
        u, sel = carry
        m = jnp.max(u, axis=1, keepdims=True)
        pos = jnp.min(jnp.where(u == m, kio, W), axis=1, keepdims=True)
        u = jnp.where(kio == pos, INT_MIN, u)
        sel = jnp.where(slot == p, pos, sel)
        return u, sel

    _, sel = lax.fori_loop(0, KS - LW, body,
                           (u, jnp.zeros((BA, LW), jnp.int32)))
    idx_ref[:, :LW] = base + slot
    idx_ref[:, LW:] = sel


@functools.lru_cache(maxsize=None)
def _make_sc_gather():
    info = plsc.get_sparse_core_info()
    nw = info.num_cores * info.num_subcores
    chunk = 128
    b_per_w = (S * KS) // nw
    nchunks = b_per_w // chunk
    mesh = plsc.VectorSubcoreMesh(core_axis_name="c", subcore_axis_name="s")

    @functools.partial(
        pl.kernel, mesh=mesh,
        out_type=jax.ShapeDtypeStruct((S * KS, DPAD), jnp.float32),
        scratch_types=[
            pltpu.VMEM((chunk,), jnp.int32),
            pltpu.VMEM((chunk, DPAD), jnp.float32),
            pltpu.SemaphoreType.DMA,
        ],
    )
    def gather(table_hbm, idx_hbm, out_hbm, idx_v, rows_v, sem):
        wid = lax.axis_index("s") * info.num_cores + lax.axis_index("c")
        base = wid * b_per_w

        def body(c, carry):
            off = base + c * chunk
            pltpu.sync_copy(idx_hbm.at[pl.ds(off, chunk)], idx_v)
            pltpu.async_copy(table_hbm.at[idx_v], rows_v, sem).wait()
            pltpu.sync_copy(rows_v, out_hbm.at[pl.ds(off, chunk)])
            return carry

        lax.fori_loop(0, nchunks, body, 0)

    return gather


def _sc_gather(cat, idx_flat):
    return _make_sc_gather()(cat, idx_flat)


def _attn_body(qabs_ref, qr_ref, sel_ref, wuv_ref, wout_ref,
               cosj_ref, sinj_ref, out_ref):
    f32 = jnp.float32
    sel = sel_ref[...].reshape(BD, KS, DPAD)
    ckv = sel[..., :DCKV]
    kr = sel[..., DCKV:DCAT]
    sp = _softplus(kr)
    mu1 = sp[..., :DR // 2]
    mu2 = sp[..., DR // 2:]
    cosj = cosj_ref[...][None]
    sinj = sinj_ref[...][None]
    keyr = jnp.concatenate(
        [mu1 * cosj - mu2 * sinj, mu1 * sinj + mu2 * cosj], axis=-1)
    wuv = wuv_ref[...]
    qabs = qabs_ref[...]
    qrr = qr_ref[...]
    s_c = lax.dot_general(qabs, ckv, (((2,), (2,)), ((0,), (0,))),
                          preferred_element_type=f32)
    s_r = lax.dot_general(qrr, keyr, (((2,), (2,)), ((0,), (0,))),
                          preferred_element_type=f32)
    s = (s_c + s_r) * SCALE
    m = jnp.max(s, axis=-1, keepdims=True)
    e = jnp.exp(s - m)
    a = e / jnp.sum(e, axis=-1, keepdims=True)
    v = lax.dot_general(a, ckv, (((2,), (1,)), ((0,), (0,))),
                        preferred_element_type=f32)
    outs = []
    for h in range(NH):
        outs.append(jnp.dot(v[:, h, :], wuv[:, h * DH:(h + 1) * DH],
                            preferred_element_type=f32))
    o = jnp.concatenate(outs, axis=1)
    out_ref[...] = jnp.dot(o, wout_ref[...], preferred_element_type=f32)


def _full(shape, dtype=jnp.float32):
    return pl.BlockSpec(shape, lambda i: (0,) * len(shape))


def kernel(x, w_dkv, w_uk, w_uv, w_dq, w_uq, w_qr, w_kr, w_out,
           wq_idx, wk_idx, w_idx, raw_delta):
    f32 = jnp.float32
    x2 = x.reshape(S, DM)

    theta = 1.0 / (10000.0 ** (2.0 * jnp.arange(0, DR // 2) / DR))
    delta = -2.0 * jnp.pi * jax.nn.sigmoid(raw_delta)
    angq = jnp.arange(S)[:, None] * theta[None, :] + delta[None, :]
    cosq, sinq = jnp.cos(angq).astype(f32), jnp.sin(angq).astype(f32)
    angj = jnp.arange(KS)[:, None] * theta[None, :] + delta[None, :]
    cosj, sinj = jnp.cos(angj).astype(f32), jnp.sin(angj).astype(f32)

    grid_a = S // BA
    ckv, xkr, qi, ki, qabs, qr = pl.pallas_call(
        _proj_body,
        grid=(grid_a,),
        in_specs=[
            pl.BlockSpec((BA, DM), lambda i: (i, 0)),
            _full((DM, DCKV)), _full((DM, DR)), _full((DM, DCQ)),
            _full((DCQ, NH * DH)), _full((DCQ, NH * DR)),
            _full((DCKV, NH * DH)), _full((DM, NH * DH)), _full((DM, DH)),
            pl.BlockSpec((BA, DR // 2), lambda i: (i, 0)),
            pl.BlockSpec((BA, DR // 2), lambda i: (i, 0)),
        ],
        out_specs=[
            pl.BlockSpec((BA, DCKV), lambda i: (i, 0)),
            pl.BlockSpec((BA, DR), lambda i: (i, 0)),
            pl.BlockSpec((BA, DM), lambda i: (i, 0)),
            pl.BlockSpec((BA, DH), lambda i: (i, 0)),
            pl.BlockSpec((BA, NH, DCKV), lambda i: (i, 0, 0)),
            pl.BlockSpec((BA, NH, DR), lambda i: (i, 0, 0)),
        ],
        out_shape=[
            jax.ShapeDtypeStruct((S, DCKV), f32),
            jax.ShapeDtypeStruct((S, DR), f32),
            jax.ShapeDtypeStruct((S, DM), f32),
            jax.ShapeDtypeStruct((S, DH), f32),
            jax.ShapeDtypeStruct((S, NH, DCKV), f32),
            jax.ShapeDtypeStruct((S, NH, DR), f32),
        ],
    )(x2, w_dkv, w_kr, w_dq, w_uq, w_qr, w_uk, wq_idx, wk_idx, cosq, sinq)

    idx_parts = []
    for blk in range(S // BA):
        W = BA * (blk + 1)
        idx_parts.append(pl.pallas_call(
            functools.partial(_topk_body, blk=blk, W=W),
            grid=(1,),
            in_specs=[
                pl.BlockSpec((BA, DM), lambda i, b=blk: (b, 0)),
                pl.BlockSpec((W, DH), lambda i: (0, 0)),
                _full((1, NH)),
            ],
            out_specs=pl.BlockSpec((BA, KS), lambda i: (0, 0)),
            out_shape=jax.ShapeDtypeStruct((BA, KS), jnp.int32),
        )(qi, ki, w_idx.reshape(1, NH)))
    idx = jnp.concatenate(idx_parts, axis=0)

    cat = jnp.concatenate(
        [ckv, xkr, jnp.zeros((S, DPAD - DCAT), f32)], axis=1)
    sel = _sc_gather(cat, idx.reshape(S * KS))

    out = pl.pallas_call(
        _attn_body,
        grid=(S // BD,),
        in_specs=[
            pl.BlockSpec((BD, NH, DCKV), lambda i: (i, 0, 0)),
            pl.BlockSpec((BD, NH, DR), lambda i: (i, 0, 0)),
            pl.BlockSpec((BD * KS, DPAD), lambda i: (i, 0)),
            _full((DCKV, NH * DH)), _full((NH * DH, DM)),
            _full((KS, DR // 2)), _full((KS, DR // 2)),
        ],
        out_specs=pl.BlockSpec((BD, DM), lambda i: (i, 0)),
        out_shape=jax.ShapeDtypeStruct((S, DM), f32),
    )(qabs, qr, sel, w_uv, w_out, cosj, sinj)

    return out.reshape(1, S, DM)
